# baseline (speedup 1.0000x reference)
import functools
import math

import jax
import jax.numpy as jnp
from jax import lax
from jax.experimental import pallas as pl
from jax.experimental.pallas import tpu as pltpu
from jax.experimental.pallas import tpu_sc as plsc

_TOPK_FRAC = 0.05


def _dot3(a, b, contract):
    dims = (contract, ((), ()))
    a_hi = a.astype(jnp.bfloat16)
    a_lo = (a - a_hi.astype(jnp.float32)).astype(jnp.bfloat16)
    b_hi = b.astype(jnp.bfloat16)
    b_lo = (b - b_hi.astype(jnp.float32)).astype(jnp.bfloat16)
    f32 = jnp.float32
    return (jax.lax.dot_general(a_hi, b_hi, dims, preferred_element_type=f32)
            + jax.lax.dot_general(a_hi, b_lo, dims, preferred_element_type=f32)
            + jax.lax.dot_general(a_lo, b_hi, dims, preferred_element_type=f32))


def _score_body(q_ref, k_ref, o_ref):
    q_bin = jnp.sign(q_ref[0]).astype(jnp.bfloat16)
    k_bin = jnp.sign(k_ref[0]).astype(jnp.bfloat16)
    o_ref[0] = jax.lax.dot_general(
        q_bin, k_bin, (((1,), (1,)), ((), ())),
        preferred_element_type=jnp.float32)


def _make_sc_select(NQ, S, D, k_top):
    NW = 32
    RB = 16
    rows_per_w = NQ // NW
    blocks_per_w = rows_per_w // RB
    NB = 129
    kf = float(k_top)

    mesh = plsc.VectorSubcoreMesh(core_axis_name="c", subcore_axis_name="s")
    MAGIC = 8388608.0

    def f2i(xf):
        return plsc.bitcast(xf + MAGIC, jnp.int32) & 0x7FFFFF

    def i2f(xi):
        return plsc.bitcast(xi | 0x4B000000, jnp.float32) - MAGIC

    @functools.partial(
        pl.kernel, mesh=mesh,
        out_type=jax.ShapeDtypeStruct((NQ,), jnp.float32),
        compiler_params=pltpu.CompilerParams(needs_layout_passes=False),
        scratch_types=[
            pltpu.VMEM((RB, S), jnp.float32),
            pltpu.VMEM((NB * 16,), jnp.float32),
            pltpu.VMEM((16,), jnp.float32),
        ],
    )
    def sc_select(scores_hbm, out_hbm, rowbuf, hist, tbuf):
        wid = lax.axis_index("s") * 2 + lax.axis_index("c")
        lane = lax.broadcasted_iota(jnp.int32, (16,), 0)
        lane_f = i2f(lane)
        zeros = jnp.zeros((16,), jnp.float32)
        ones = jnp.ones((16,), jnp.float32)

        def block_fn(b, _):
            rbase = wid * rows_per_w + b * RB
            pltpu.sync_copy(scores_hbm.at[pl.ds(rbase, RB)], rowbuf)

            def clr(i, c):
                hist[pl.ds(i * 16, 16)] = zeros
                return c
            lax.fori_loop(0, NB, clr, 0)

            U = 8

            def p1(g, carry):
                jcol = carry
                for jj in range(U):
                    x = plsc.load_gather(rowbuf, [lane, jcol + jj])
                    idx = f2i(x * 16.0 + (float(D) * 16.0 + lane_f))
                    plsc.addupdate_scatter(hist, [idx], ones)
                return jcol + U
            lax.fori_loop(0, S // U, p1, jnp.zeros((16,), jnp.int32))

            def scan_fn(i, carry):
                acc, s_star, n_gt, found, bkt_f = carry
                bkt = NB - 1 - i
                h_b = hist[pl.ds(bkt * 16, 16)]
                acc2 = acc + h_b
                newly = (acc2 >= kf) & (found == 0.0)
                s_star = jnp.where(newly, bkt_f, s_star)
                n_gt = jnp.where(newly, acc, n_gt)
                found = jnp.where(newly, ones, found)
                return acc2, s_star, n_gt, found, bkt_f - 1.0
            _, s_star, n_gt, _, _ = lax.fori_loop(
                0, NB, scan_fn,
                (zeros, zeros, zeros, zeros,
                 jnp.full((16,), float(NB - 1), jnp.float32)),
                unroll=4)
            s_star_sc = s_star - float(D)
            need = kf - n_gt

            def p2(g, carry):
                run, idx_cut, j_f, jcol = carry
                for jj in range(U):
                    x = plsc.load_gather(rowbuf, [lane, jcol + jj])
                    m = x == s_star_sc
                    run = run + jnp.where(m, 1.0, 0.0)
                    hit = m & (run == need)
                    idx_cut = jnp.where(hit, j_f + float(jj), idx_cut)
                return run, idx_cut, j_f + float(U), jcol + U
            _, idx_cut, _, _ = lax.fori_loop(
                0, S // U, p2,
                (zeros, zeros, zeros, jnp.zeros((16,), jnp.int32)))

            tbuf[...] = (s_star_sc * float(S)
                         + (float(S - 1) - idx_cut))
            pltpu.sync_copy(tbuf, out_hbm.at[pl.ds(rbase, RB)])
            return 0

        lax.fori_loop(0, blocks_per_w, block_fn, 0)

    return sc_select


def _attn_body(q_ref, k_ref, v_ref, t_ref, o_ref, *, S, D):
    q = q_ref[0]
    k = k_ref[0]
    v = v_ref[0]
    thr = t_ref[0]

    q_bin = jnp.sign(q).astype(jnp.bfloat16)
    k_bin = jnp.sign(k).astype(jnp.bfloat16)
    s_int = jax.lax.dot_general(
        q_bin, k_bin, (((1,), (1,)), ((), ())),
        preferred_element_type=jnp.float32)
    col = jax.lax.broadcasted_iota(jnp.int32, s_int.shape, 1).astype(
        jnp.float32)
    ckey = s_int * float(S) + (float(S - 1) - col)
    mask = ckey >= thr

    ps = _dot3(q, k, ((1,), (1,))) * (1.0 / math.sqrt(D))
    psm = jnp.where(mask, ps, -jnp.inf)
    m = jnp.max(psm, axis=1, keepdims=True)
    e = jnp.exp(psm - m)
    denom = jnp.sum(e, axis=1, keepdims=True)
    out = _dot3(e, v, ((1,), (0,))) / denom
    o_ref[0] = out


def kernel(q, k, v):
    B, H, S, D = q.shape
    k_top = max(1, int(S * _TOPK_FRAC))
    BQ = 256
    BH = B * H
    NQ = BH * S
    qr = q.reshape(BH, S, D)
    kr = k.reshape(BH, S, D)
    vr = v.reshape(BH, S, D)
    grid = (BH, S // BQ)

    scores = pl.pallas_call(
        _score_body,
        grid=grid,
        in_specs=[
            pl.BlockSpec((1, BQ, D), lambda h, i: (h, i, 0)),
            pl.BlockSpec((1, S, D), lambda h, i: (h, 0, 0)),
        ],
        out_specs=pl.BlockSpec((1, BQ, S), lambda h, i: (h, i, 0)),
        out_shape=jax.ShapeDtypeStruct((BH, S, S), jnp.float32),
        compiler_params=pltpu.CompilerParams(
            dimension_semantics=("arbitrary", "arbitrary")),
    )(qr, kr)

    thr = _make_sc_select(NQ, S, D, k_top)(scores.reshape(NQ, S))
    thr3 = thr.reshape(BH * (S // BQ), BQ, 1)

    out = pl.pallas_call(
        functools.partial(_attn_body, S=S, D=D),
        grid=grid,
        in_specs=[
            pl.BlockSpec((1, BQ, D), lambda h, i: (h, i, 0)),
            pl.BlockSpec((1, S, D), lambda h, i: (h, 0, 0)),
            pl.BlockSpec((1, S, D), lambda h, i: (h, 0, 0)),
            pl.BlockSpec((1, BQ, 1),
                         lambda h, i: (h * (S // BQ) + i, 0, 0)),
        ],
        out_specs=pl.BlockSpec((1, BQ, D), lambda h, i: (h, i, 0)),
        out_shape=jax.ShapeDtypeStruct((BH, S, D), jnp.float32),
        compiler_params=pltpu.CompilerParams(
            dimension_semantics=("arbitrary", "arbitrary")),
    )(qr, kr, vr, thr3)
    return out.reshape(B, H, S, D)


# speedup vs baseline: 1.6278x; 1.6278x over previous
import functools
import math

import jax
import jax.numpy as jnp
from jax import lax
from jax.experimental import pallas as pl
from jax.experimental.pallas import tpu as pltpu
from jax.experimental.pallas import tpu_sc as plsc

_TOPK_FRAC = 0.05


def _dot3(a, b, contract):
    dims = (contract, ((), ()))
    a_hi = a.astype(jnp.bfloat16)
    a_lo = (a - a_hi.astype(jnp.float32)).astype(jnp.bfloat16)
    b_hi = b.astype(jnp.bfloat16)
    b_lo = (b - b_hi.astype(jnp.float32)).astype(jnp.bfloat16)
    f32 = jnp.float32
    return (jax.lax.dot_general(a_hi, b_hi, dims, preferred_element_type=f32)
            + jax.lax.dot_general(a_hi, b_lo, dims, preferred_element_type=f32)
            + jax.lax.dot_general(a_lo, b_hi, dims, preferred_element_type=f32))


def _score_body(q_ref, k_ref, o_ref):
    q_bin = jnp.sign(q_ref[0]).astype(jnp.bfloat16)
    k_bin = jnp.sign(k_ref[0]).astype(jnp.bfloat16)
    o_ref[0] = jax.lax.dot_general(
        q_bin, k_bin, (((1,), (1,)), ((), ())),
        preferred_element_type=jnp.float32)


def _make_sc_select(NQ, S, D, k_top):
    NW = 32
    RB = 16
    rows_per_w = NQ // NW
    blocks_per_w = rows_per_w // RB
    NB = 129
    kf = float(k_top)

    mesh = plsc.VectorSubcoreMesh(core_axis_name="c", subcore_axis_name="s")
    MAGIC = 8388608.0

    def f2i(xf):
        return plsc.bitcast(xf + MAGIC, jnp.int32) & 0x7FFFFF

    def i2f(xi):
        return plsc.bitcast(xi | 0x4B000000, jnp.float32) - MAGIC

    @functools.partial(
        pl.kernel, mesh=mesh,
        out_type=jax.ShapeDtypeStruct((NQ,), jnp.float32),
        compiler_params=pltpu.CompilerParams(needs_layout_passes=False),
        scratch_types=[
            pltpu.VMEM((RB, S), jnp.float32),
            pltpu.VMEM((NB * 16,), jnp.float32),
            pltpu.VMEM((16,), jnp.float32),
        ],
    )
    def sc_select(scores_hbm, out_hbm, rowbuf, hist, tbuf):
        wid = lax.axis_index("s") * 2 + lax.axis_index("c")
        lane = lax.broadcasted_iota(jnp.int32, (16,), 0)
        lane_f = i2f(lane)
        zeros = jnp.zeros((16,), jnp.float32)
        ones = jnp.ones((16,), jnp.float32)

        def block_fn(b, _):
            rbase = wid * rows_per_w + b * RB
            pltpu.sync_copy(scores_hbm.at[pl.ds(rbase, RB)], rowbuf)

            def clr(i, c):
                hist[pl.ds(i * 16, 16)] = zeros
                return c
            lax.fori_loop(0, NB, clr, 0)

            @plsc.parallel_loop(0, S, unroll=8,
                                carry=jnp.zeros((16,), jnp.int32))
            def _p1(i, jcol):
                x = plsc.load_gather(rowbuf, [lane, jcol])
                idx = f2i(x * 16.0 + (float(D) * 16.0 + lane_f))
                plsc.addupdate_scatter(hist, [idx], ones)
                return jcol + 1

            def scan_fn(i, carry):
                acc, s_star, n_gt, found, bkt_f = carry
                bkt = NB - 1 - i
                h_b = hist[pl.ds(bkt * 16, 16)]
                acc2 = acc + h_b
                newly = (acc2 >= kf) & (found == 0.0)
                s_star = jnp.where(newly, bkt_f, s_star)
                n_gt = jnp.where(newly, acc, n_gt)
                found = jnp.where(newly, ones, found)
                return acc2, s_star, n_gt, found, bkt_f - 1.0
            _, s_star, n_gt, _, _ = lax.fori_loop(
                0, NB, scan_fn,
                (zeros, zeros, zeros, zeros,
                 jnp.full((16,), float(NB - 1), jnp.float32)),
                unroll=4)
            s_star_sc = s_star - float(D)
            need = kf - n_gt

            @plsc.parallel_loop(0, S, unroll=8,
                                carry=(zeros, zeros, zeros,
                                       jnp.zeros((16,), jnp.int32)))
            def _p2(i, carry):
                run, idx_cut, j_f, jcol = carry
                x = plsc.load_gather(rowbuf, [lane, jcol])
                m = x == s_star_sc
                run2 = run + jnp.where(m, 1.0, 0.0)
                hit = m & (run2 == need)
                idx_cut = jnp.where(hit, j_f, idx_cut)
                return run2, idx_cut, j_f + 1.0, jcol + 1
            _, idx_cut, _, _ = _p2

            tbuf[...] = (s_star_sc * float(S)
                         + (float(S - 1) - idx_cut))
            pltpu.sync_copy(tbuf, out_hbm.at[pl.ds(rbase, RB)])
            return 0

        lax.fori_loop(0, blocks_per_w, block_fn, 0)

    return sc_select


def _attn_body(q_ref, k_ref, v_ref, t_ref, o_ref, *, S, D):
    q = q_ref[0]
    k = k_ref[0]
    v = v_ref[0]
    thr = t_ref[0]

    q_bin = jnp.sign(q).astype(jnp.bfloat16)
    k_bin = jnp.sign(k).astype(jnp.bfloat16)
    s_int = jax.lax.dot_general(
        q_bin, k_bin, (((1,), (1,)), ((), ())),
        preferred_element_type=jnp.float32)
    col = jax.lax.broadcasted_iota(jnp.int32, s_int.shape, 1).astype(
        jnp.float32)
    ckey = s_int * float(S) + (float(S - 1) - col)
    mask = ckey >= thr

    ps = _dot3(q, k, ((1,), (1,))) * (1.0 / math.sqrt(D))
    psm = jnp.where(mask, ps, -jnp.inf)
    m = jnp.max(psm, axis=1, keepdims=True)
    e = jnp.exp(psm - m)
    denom = jnp.sum(e, axis=1, keepdims=True)
    out = _dot3(e, v, ((1,), (0,))) / denom
    o_ref[0] = out


def kernel(q, k, v):
    B, H, S, D = q.shape
    k_top = max(1, int(S * _TOPK_FRAC))
    BQ = 256
    BH = B * H
    NQ = BH * S
    qr = q.reshape(BH, S, D)
    kr = k.reshape(BH, S, D)
    vr = v.reshape(BH, S, D)
    grid = (BH, S // BQ)

    scores = pl.pallas_call(
        _score_body,
        grid=grid,
        in_specs=[
            pl.BlockSpec((1, BQ, D), lambda h, i: (h, i, 0)),
            pl.BlockSpec((1, S, D), lambda h, i: (h, 0, 0)),
        ],
        out_specs=pl.BlockSpec((1, BQ, S), lambda h, i: (h, i, 0)),
        out_shape=jax.ShapeDtypeStruct((BH, S, S), jnp.float32),
        compiler_params=pltpu.CompilerParams(
            dimension_semantics=("arbitrary", "arbitrary")),
    )(qr, kr)

    thr = _make_sc_select(NQ, S, D, k_top)(scores.reshape(NQ, S))
    thr3 = thr.reshape(BH * (S // BQ), BQ, 1)

    out = pl.pallas_call(
        functools.partial(_attn_body, S=S, D=D),
        grid=grid,
        in_specs=[
            pl.BlockSpec((1, BQ, D), lambda h, i: (h, i, 0)),
            pl.BlockSpec((1, S, D), lambda h, i: (h, 0, 0)),
            pl.BlockSpec((1, S, D), lambda h, i: (h, 0, 0)),
            pl.BlockSpec((1, BQ, 1),
                         lambda h, i: (h * (S // BQ) + i, 0, 0)),
        ],
        out_specs=pl.BlockSpec((1, BQ, D), lambda h, i: (h, i, 0)),
        out_shape=jax.ShapeDtypeStruct((BH, S, D), jnp.float32),
        compiler_params=pltpu.CompilerParams(
            dimension_semantics=("arbitrary", "arbitrary")),
    )(qr, kr, vr, thr3)
    return out.reshape(B, H, S, D)


# speedup vs baseline: 1.6711x; 1.0266x over previous
import functools
import math

import jax
import jax.numpy as jnp
from jax import lax
from jax.experimental import pallas as pl
from jax.experimental.pallas import tpu as pltpu
from jax.experimental.pallas import tpu_sc as plsc

_TOPK_FRAC = 0.05


def _dot3(a, b, contract):
    dims = (contract, ((), ()))
    a_hi = a.astype(jnp.bfloat16)
    a_lo = (a - a_hi.astype(jnp.float32)).astype(jnp.bfloat16)
    b_hi = b.astype(jnp.bfloat16)
    b_lo = (b - b_hi.astype(jnp.float32)).astype(jnp.bfloat16)
    f32 = jnp.float32
    return (jax.lax.dot_general(a_hi, b_hi, dims, preferred_element_type=f32)
            + jax.lax.dot_general(a_hi, b_lo, dims, preferred_element_type=f32)
            + jax.lax.dot_general(a_lo, b_hi, dims, preferred_element_type=f32))


def _score_body(q_ref, k_ref, o_ref):
    q_bin = jnp.sign(q_ref[0]).astype(jnp.bfloat16)
    k_bin = jnp.sign(k_ref[0]).astype(jnp.bfloat16)
    o_ref[0] = jax.lax.dot_general(
        q_bin, k_bin, (((1,), (1,)), ((), ())),
        preferred_element_type=jnp.float32)


def _make_sc_select(NQ, S, D, k_top):
    NW = 32
    RB = 16
    rows_per_w = NQ // NW
    blocks_per_w = rows_per_w // RB
    NB = 129
    kf = float(k_top)

    mesh = plsc.VectorSubcoreMesh(core_axis_name="c", subcore_axis_name="s")
    MAGIC = 8388608.0

    def f2i(xf):
        return plsc.bitcast(xf + MAGIC, jnp.int32) & 0x7FFFFF

    def i2f(xi):
        return plsc.bitcast(xi | 0x4B000000, jnp.float32) - MAGIC

    @functools.partial(
        pl.kernel, mesh=mesh,
        out_type=jax.ShapeDtypeStruct((NQ,), jnp.float32),
        compiler_params=pltpu.CompilerParams(needs_layout_passes=False),
        scratch_types=[
            pltpu.VMEM((RB, S), jnp.float32),
            pltpu.VMEM((NB * 16,), jnp.float32),
            pltpu.VMEM((16,), jnp.float32),
        ],
    )
    def sc_select(scores_hbm, out_hbm, rowbuf, hist, tbuf):
        wid = lax.axis_index("s") * 2 + lax.axis_index("c")
        lane = lax.broadcasted_iota(jnp.int32, (16,), 0)
        lane_f = i2f(lane)
        zeros = jnp.zeros((16,), jnp.float32)
        ones = jnp.ones((16,), jnp.float32)

        def block_fn(b, _):
            rbase = wid * rows_per_w + b * RB
            pltpu.sync_copy(scores_hbm.at[pl.ds(rbase, RB)], rowbuf)

            def clr(i, c):
                hist[pl.ds(i * 16, 16)] = zeros
                return c
            lax.fori_loop(0, NB, clr, 0)

            @plsc.parallel_loop(0, S, unroll=8,
                                carry=jnp.zeros((16,), jnp.int32))
            def _p1(i, jcol):
                x = plsc.load_gather(rowbuf, [lane, jcol])
                idx = f2i(x * 16.0 + (float(D) * 16.0 + lane_f))
                plsc.addupdate_scatter(hist, [idx], ones)
                return jcol + 1

            def scan_fn(i, carry):
                acc, s_star, n_gt, found, bkt_f = carry
                bkt = NB - 1 - i
                h_b = hist[pl.ds(bkt * 16, 16)]
                acc2 = acc + h_b
                newly = (acc2 >= kf) & (found == 0.0)
                s_star = jnp.where(newly, bkt_f, s_star)
                n_gt = jnp.where(newly, acc, n_gt)
                found = jnp.where(newly, ones, found)
                return acc2, s_star, n_gt, found, bkt_f - 1.0
            _, s_star, n_gt, _, _ = lax.fori_loop(
                0, NB, scan_fn,
                (zeros, zeros, zeros, zeros,
                 jnp.full((16,), float(NB - 1), jnp.float32)),
                unroll=4)
            s_star_sc = s_star - float(D)
            need = kf - n_gt

            @plsc.parallel_loop(0, S, unroll=8,
                                carry=(zeros, zeros, zeros,
                                       jnp.zeros((16,), jnp.int32)))
            def _p2(i, carry):
                run, idx_cut, j_f, jcol = carry
                x = plsc.load_gather(rowbuf, [lane, jcol])
                m = x == s_star_sc
                run2 = run + jnp.where(m, 1.0, 0.0)
                hit = m & (run2 == need)
                idx_cut = jnp.where(hit, j_f, idx_cut)
                return run2, idx_cut, j_f + 1.0, jcol + 1
            _, idx_cut, _, _ = _p2

            tbuf[...] = (s_star_sc * float(S)
                         + (float(S - 1) - idx_cut))
            pltpu.sync_copy(tbuf, out_hbm.at[pl.ds(rbase, RB)])
            return 0

        lax.fori_loop(0, blocks_per_w, block_fn, 0)

    return sc_select


def _attn_body(q_ref, k_ref, v_ref, t_ref, o_ref, *, S, D):
    q = q_ref[0]
    k = k_ref[0]
    v = v_ref[0]
    thr = t_ref[0]

    q_bin = jnp.sign(q).astype(jnp.bfloat16)
    k_bin = jnp.sign(k).astype(jnp.bfloat16)
    s_int = jax.lax.dot_general(
        q_bin, k_bin, (((1,), (1,)), ((), ())),
        preferred_element_type=jnp.float32)
    col = jax.lax.broadcasted_iota(jnp.int32, s_int.shape, 1).astype(
        jnp.float32)
    ckey = s_int * float(S) + (float(S - 1) - col)
    mask = ckey >= thr

    ps = _dot3(q, k, ((1,), (1,))) * (1.0 / math.sqrt(D))
    psm = jnp.where(mask, ps, -jnp.inf)
    m = jnp.max(psm, axis=1, keepdims=True)
    e = jnp.exp(psm - m)
    denom = jnp.sum(e, axis=1, keepdims=True)
    out = _dot3(e, v, ((1,), (0,))) / denom
    o_ref[0] = out


def kernel(q, k, v):
    B, H, S, D = q.shape
    k_top = max(1, int(S * _TOPK_FRAC))
    BQ = 256
    BH = B * H
    NQ = BH * S
    qr = q.reshape(BH, S, D)
    kr = k.reshape(BH, S, D)
    vr = v.reshape(BH, S, D)
    grid = (BH, S // BQ)

    scores = pl.pallas_call(
        _score_body,
        grid=grid,
        in_specs=[
            pl.BlockSpec((1, BQ, D), lambda h, i: (h, i, 0)),
            pl.BlockSpec((1, S, D), lambda h, i: (h, 0, 0)),
        ],
        out_specs=pl.BlockSpec((1, BQ, S), lambda h, i: (h, i, 0)),
        out_shape=jax.ShapeDtypeStruct((BH, S, S), jnp.float32),
        compiler_params=pltpu.CompilerParams(
            dimension_semantics=("arbitrary", "arbitrary")),
    )(qr, kr)

    HH = BH // 2
    s2 = scores.reshape(NQ, S)
    sc_sel = _make_sc_select(NQ // 2, S, D, k_top)
    thr_a = sc_sel(s2[: NQ // 2])
    thr_b = sc_sel(s2[NQ // 2:])

    def attn_half(qh, kh, vh, thr):
        gridh = (HH, S // BQ)
        thr3 = thr.reshape(HH * (S // BQ), BQ, 1)
        return pl.pallas_call(
            functools.partial(_attn_body, S=S, D=D),
            grid=gridh,
            in_specs=[
                pl.BlockSpec((1, BQ, D), lambda h, i: (h, i, 0)),
                pl.BlockSpec((1, S, D), lambda h, i: (h, 0, 0)),
                pl.BlockSpec((1, S, D), lambda h, i: (h, 0, 0)),
                pl.BlockSpec((1, BQ, 1),
                             lambda h, i: (h * (S // BQ) + i, 0, 0)),
            ],
            out_specs=pl.BlockSpec((1, BQ, D), lambda h, i: (h, i, 0)),
            out_shape=jax.ShapeDtypeStruct((HH, S, D), jnp.float32),
            compiler_params=pltpu.CompilerParams(
                dimension_semantics=("arbitrary", "arbitrary")),
        )(qh, kh, vh, thr3)

    out_a = attn_half(qr[:HH], kr[:HH], vr[:HH], thr_a)
    out_b = attn_half(qr[HH:], kr[HH:], vr[HH:], thr_b)
    out = jnp.concatenate([out_a, out_b], axis=0)
    return out.reshape(B, H, S, D)


# speedup vs baseline: 2.3025x; 1.3778x over previous
import functools
import math

import jax
import jax.numpy as jnp
from jax import lax
from jax.experimental import pallas as pl
from jax.experimental.pallas import tpu as pltpu
from jax.experimental.pallas import tpu_sc as plsc

_TOPK_FRAC = 0.05


def _dot3(a, b, contract):
    dims = (contract, ((), ()))
    a_hi = a.astype(jnp.bfloat16)
    a_lo = (a - a_hi.astype(jnp.float32)).astype(jnp.bfloat16)
    b_hi = b.astype(jnp.bfloat16)
    b_lo = (b - b_hi.astype(jnp.float32)).astype(jnp.bfloat16)
    f32 = jnp.float32
    return (jax.lax.dot_general(a_hi, b_hi, dims, preferred_element_type=f32)
            + jax.lax.dot_general(a_hi, b_lo, dims, preferred_element_type=f32)
            + jax.lax.dot_general(a_lo, b_hi, dims, preferred_element_type=f32))


def _score_body(q_ref, k_ref, o_ref):
    q_bin = jnp.sign(q_ref[0]).astype(jnp.bfloat16)
    k_bin = jnp.sign(k_ref[0]).astype(jnp.bfloat16)
    o_ref[0] = jax.lax.dot_general(
        q_bin, k_bin, (((1,), (1,)), ((), ())),
        preferred_element_type=jnp.float32)


def _make_sc_select(NQ, S, D, k_top):
    NW = 32
    RB = 16
    rows_per_w = NQ // NW
    blocks_per_w = rows_per_w // RB
    NB = 129
    kf = float(k_top)

    mesh = plsc.VectorSubcoreMesh(core_axis_name="c", subcore_axis_name="s")
    MAGIC = 8388608.0

    def f2i(xf):
        return plsc.bitcast(xf + MAGIC, jnp.int32) & 0x7FFFFF

    def i2f(xi):
        return plsc.bitcast(xi | 0x4B000000, jnp.float32) - MAGIC

    NQTR = 4
    QW = S // NQTR

    @functools.partial(
        pl.kernel, mesh=mesh,
        out_type=jax.ShapeDtypeStruct((NQ,), jnp.float32),
        compiler_params=pltpu.CompilerParams(needs_layout_passes=False),
        scratch_types=[
            pltpu.VMEM((RB, S), jnp.float32),
            pltpu.VMEM((NB * NQTR * 16,), jnp.float32),
            pltpu.VMEM((16,), jnp.float32),
        ],
    )
    def sc_select(scores_hbm, out_hbm, rowbuf, hist, tbuf):
        wid = lax.axis_index("s") * 2 + lax.axis_index("c")
        lane = lax.broadcasted_iota(jnp.int32, (16,), 0)
        lane_f = i2f(lane)
        zeros = jnp.zeros((16,), jnp.float32)
        ones = jnp.ones((16,), jnp.float32)

        def block_fn(b, _):
            rbase = wid * rows_per_w + b * RB
            pltpu.sync_copy(scores_hbm.at[pl.ds(rbase, RB)], rowbuf)

            @plsc.parallel_loop(0, NB * NQTR, unroll=8)
            def _clr(i):
                hist[pl.ds(i * 16, 16)] = zeros

            for qq in range(NQTR):
                cq = (float(D * NQTR * 16 + qq * 16) + lane_f + MAGIC)

                @plsc.parallel_loop(0, QW, unroll=8,
                                    carry=jnp.full((16,), qq * QW,
                                                   jnp.int32))
                def _p1(i, jcol, cq=cq):
                    x = plsc.load_gather(rowbuf, [lane, jcol])
                    idx = plsc.bitcast(x * float(NQTR * 16) + cq,
                                       jnp.int32) & 0x7FFFFF
                    plsc.addupdate_scatter(hist, [idx], ones)
                    return jcol + 1

            def scan_fn(i, carry):
                acc, s_star, n_gt, found, bkt_f = carry
                bkt = NB - 1 - i
                h_b = (hist[pl.ds(bkt * (NQTR * 16), 16)]
                       + hist[pl.ds(bkt * (NQTR * 16) + 16, 16)]
                       + hist[pl.ds(bkt * (NQTR * 16) + 32, 16)]
                       + hist[pl.ds(bkt * (NQTR * 16) + 48, 16)])
                acc2 = acc + h_b
                newly = (acc2 >= kf) & (found == 0.0)
                s_star = jnp.where(newly, bkt_f, s_star)
                n_gt = jnp.where(newly, acc, n_gt)
                found = jnp.where(newly, ones, found)
                return acc2, s_star, n_gt, found, bkt_f - 1.0
            _, s_star, n_gt, _, _ = lax.fori_loop(
                0, NB, scan_fn,
                (zeros, zeros, zeros, zeros,
                 jnp.full((16,), float(NB - 1), jnp.float32)),
                unroll=4)
            s_star_sc = s_star - float(D)
            need = kf - n_gt

            base = lax.shift_left(f2i(s_star), 6) + lane
            q0 = plsc.load_gather(hist, [base])
            q1 = plsc.load_gather(hist, [base + 16])
            q2 = plsc.load_gather(hist, [base + 32])
            c1 = q0 + q1
            c2 = c1 + q2
            ge0 = q0 >= need
            ge1 = c1 >= need
            ge2 = c2 >= need
            qf = jnp.where(ge0, 0.0,
                           jnp.where(ge1, 1.0, jnp.where(ge2, 2.0, 3.0)))
            sub = jnp.where(ge0, zeros,
                            jnp.where(ge1, q0, jnp.where(ge2, c1, c2)))
            need2 = need - sub
            jf0 = qf * float(QW)
            jcol0 = f2i(jf0)

            @plsc.parallel_loop(0, QW, unroll=8,
                                carry=(zeros, zeros, jf0, jcol0))
            def _p2(i, carry):
                run, idx_cut, j_f, jcol = carry
                x = plsc.load_gather(rowbuf, [lane, jcol])
                m = x == s_star_sc
                run2 = run + jnp.where(m, 1.0, 0.0)
                hit = m & (run2 == need2)
                idx_cut = jnp.where(hit, j_f, idx_cut)
                return run2, idx_cut, j_f + 1.0, jcol + 1
            _, idx_cut, _, _ = _p2

            tbuf[...] = (s_star_sc * float(S)
                         + (float(S - 1) - idx_cut))
            pltpu.sync_copy(tbuf, out_hbm.at[pl.ds(rbase, RB)])
            return 0

        lax.fori_loop(0, blocks_per_w, block_fn, 0)

    return sc_select


def _attn_body(q_ref, k_ref, v_ref, t_ref, o_ref, *, S, D):
    q = q_ref[0]
    k = k_ref[0]
    v = v_ref[0]
    thr = t_ref[0]

    q_bin = jnp.sign(q).astype(jnp.bfloat16)
    k_bin = jnp.sign(k).astype(jnp.bfloat16)
    s_int = jax.lax.dot_general(
        q_bin, k_bin, (((1,), (1,)), ((), ())),
        preferred_element_type=jnp.float32)
    col = jax.lax.broadcasted_iota(jnp.int32, s_int.shape, 1).astype(
        jnp.float32)
    ckey = s_int * float(S) + (float(S - 1) - col)
    mask = ckey >= thr

    ps = _dot3(q, k, ((1,), (1,))) * (1.0 / math.sqrt(D))
    psm = jnp.where(mask, ps, -jnp.inf)
    m = jnp.max(psm, axis=1, keepdims=True)
    e = jnp.exp(psm - m)
    denom = jnp.sum(e, axis=1, keepdims=True)
    out = _dot3(e, v, ((1,), (0,))) / denom
    o_ref[0] = out


def kernel(q, k, v):
    B, H, S, D = q.shape
    k_top = max(1, int(S * _TOPK_FRAC))
    BQ = 256
    BH = B * H
    NQ = BH * S
    qr = q.reshape(BH, S, D)
    kr = k.reshape(BH, S, D)
    vr = v.reshape(BH, S, D)
    grid = (BH, S // BQ)

    scores = pl.pallas_call(
        _score_body,
        grid=grid,
        in_specs=[
            pl.BlockSpec((1, BQ, D), lambda h, i: (h, i, 0)),
            pl.BlockSpec((1, S, D), lambda h, i: (h, 0, 0)),
        ],
        out_specs=pl.BlockSpec((1, BQ, S), lambda h, i: (h, i, 0)),
        out_shape=jax.ShapeDtypeStruct((BH, S, S), jnp.float32),
        compiler_params=pltpu.CompilerParams(
            dimension_semantics=("arbitrary", "arbitrary")),
    )(qr, kr)

    HH = BH // 2
    s2 = scores.reshape(NQ, S)
    sc_sel = _make_sc_select(NQ // 2, S, D, k_top)
    thr_a = sc_sel(s2[: NQ // 2])
    thr_b = sc_sel(s2[NQ // 2:])

    def attn_half(qh, kh, vh, thr):
        gridh = (HH, S // BQ)
        thr3 = thr.reshape(HH * (S // BQ), BQ, 1)
        return pl.pallas_call(
            functools.partial(_attn_body, S=S, D=D),
            grid=gridh,
            in_specs=[
                pl.BlockSpec((1, BQ, D), lambda h, i: (h, i, 0)),
                pl.BlockSpec((1, S, D), lambda h, i: (h, 0, 0)),
                pl.BlockSpec((1, S, D), lambda h, i: (h, 0, 0)),
                pl.BlockSpec((1, BQ, 1),
                             lambda h, i: (h * (S // BQ) + i, 0, 0)),
            ],
            out_specs=pl.BlockSpec((1, BQ, D), lambda h, i: (h, i, 0)),
            out_shape=jax.ShapeDtypeStruct((HH, S, D), jnp.float32),
            compiler_params=pltpu.CompilerParams(
                dimension_semantics=("arbitrary", "arbitrary")),
        )(qh, kh, vh, thr3)

    out_a = attn_half(qr[:HH], kr[:HH], vr[:HH], thr_a)
    out_b = attn_half(qr[HH:], kr[HH:], vr[HH:], thr_b)
    out = jnp.concatenate([out_a, out_b], axis=0)
    return out.reshape(B, H, S, D)
